# Initial kernel scaffold; baseline (speedup 1.0000x reference)
import jax
import jax.numpy as jnp
from jax.experimental import pallas as pl


def kernel(x, edge_index, W1, b1, W2, b2, W3, b3):
    raise NotImplementedError("write your pallas kernel here")


# speedup vs baseline: 7.9098x; 7.9098x over previous
import functools

import jax
import jax.numpy as jnp
from jax import lax
from jax.experimental import pallas as pl
from jax.experimental.pallas import tpu as pltpu
from jax.experimental.pallas import tpu_sc as plsc

EW = 125
MB = 2000


def _dinv16(degS, n):
    def body(d_ref, out_ref):
        out_ref[...] = 1.0 / jnp.sqrt(d_ref[0, :, :16])

    return pl.pallas_call(
        body,
        grid=(n // MB,),
        in_specs=[pl.BlockSpec((1, MB, 128), lambda m: (0, m, 0))],
        out_specs=pl.BlockSpec((MB, 16), lambda m: (m, 0)),
        out_shape=jax.ShapeDtypeStruct((n, 16), jnp.float32),
    )(degS)


def _matmul_g(h, W, dinv16, n_pad):
    n, fin = h.shape
    fout = W.shape[1]
    cin, cout = fin // 128, fout // 128

    def body(h_ref, w_ref, dinv_ref, out_ref):
        ci = pl.program_id(2)

        @pl.when(ci == 0)
        def _():
            out_ref[...] = jnp.zeros(out_ref.shape, out_ref.dtype)

        out_ref[0] += jnp.dot(h_ref[...], w_ref[...],
                              preferred_element_type=jnp.float32)

        @pl.when(ci == cin - 1)
        def _():
            out_ref[0] = out_ref[0] * dinv_ref[:, :1]

    return pl.pallas_call(
        body,
        grid=(n // MB, cout, cin),
        in_specs=[
            pl.BlockSpec((MB, 128), lambda m, co, ci: (m, ci)),
            pl.BlockSpec((128, 128), lambda m, co, ci: (ci, co)),
            pl.BlockSpec((MB, 16), lambda m, co, ci: (m, 0)),
        ],
        out_specs=pl.BlockSpec((1, MB, 128), lambda m, co, ci: (co, m, 0)),
        out_shape=jax.ShapeDtypeStruct((cout, n_pad, 128), jnp.float32),
        compiler_params=pltpu.CompilerParams(
            dimension_semantics=("parallel", "parallel", "arbitrary")),
    )(h, W, dinv16)


def _activate(S, dinv16, b2d, n):
    c = S.shape[0]

    def body(s_ref, dinv_ref, b_ref, out_ref):
        out_ref[...] = jnp.tanh(s_ref[0] * dinv_ref[:, :1] + b_ref[0])

    return pl.pallas_call(
        body,
        grid=(n // MB, c),
        in_specs=[
            pl.BlockSpec((1, MB, 128), lambda m, c_: (c_, m, 0)),
            pl.BlockSpec((MB, 16), lambda m, c_: (m, 0)),
            pl.BlockSpec((1, 1, 128), lambda m, c_: (c_, 0, 0)),
        ],
        out_specs=pl.BlockSpec((MB, 128), lambda m, c_: (m, c_)),
        out_shape=jax.ShapeDtypeStruct((n, c * 128), jnp.float32),
    )(S, dinv16, b2d)


def _sc_edge(g, src2, dst2):
    c, n_pad, _ = g.shape
    nrows, ew = src2.shape
    rows_per_tile = nrows // 16
    rpt_n = n_pad // 16
    mesh = plsc.VectorSubcoreMesh(core_axis_name="c", subcore_axis_name="s", num_cores=2, num_subcores=16)

    @functools.partial(
        pl.kernel,
        out_type=jax.ShapeDtypeStruct((c, n_pad, 128), jnp.float32),
        mesh=mesh,
        scratch_types=[
            pltpu.VMEM((rows_per_tile, ew), jnp.int32),
            pltpu.VMEM((rows_per_tile, ew), jnp.int32),
            pltpu.VMEM((ew, 128), jnp.float32),
            pltpu.VMEM_SHARED((n_pad, 128), jnp.float32),
        ],
    )
    def k(g_hbm, src_hbm, dst_hbm, out_hbm, src_v, dst_v, rows_v, s_sh):
        cid = lax.axis_index("c")
        sid = lax.axis_index("s")
        er0 = sid * rows_per_tile
        pltpu.sync_copy(src_hbm.at[pl.ds(er0, rows_per_tile)], src_v)
        pltpu.sync_copy(dst_hbm.at[pl.ds(er0, rows_per_tile)], dst_v)
        nb = sid * rpt_n
        for chunk in range(c):

            @pl.when(chunk % 2 == cid)
            def _(chunk=chunk):
                g_c = g_hbm.at[chunk]
                pltpu.sync_copy(g_c.at[pl.ds(nb, rpt_n)],
                                s_sh.at[pl.ds(nb, rpt_n)])
                plsc.subcore_barrier()

                def body(j, carry):
                    pltpu.sync_copy(g_c.at[src_v.at[j]], rows_v)
                    pltpu.sync_copy(rows_v, s_sh.at[dst_v.at[j]], add=True)
                    return carry

                lax.fori_loop(0, rows_per_tile, body, 0)
                plsc.subcore_barrier()
                pltpu.sync_copy(s_sh.at[pl.ds(nb, rpt_n)],
                                out_hbm.at[chunk, pl.ds(nb, rpt_n)])
                plsc.subcore_barrier()

    return k(g, src2, dst2)


def kernel(x, edge_index, W1, b1, W2, b2, W3, b3):
    n = x.shape[0]
    e = edge_index.shape[1]
    n_pad = -(-n // 128) * 128
    src2 = edge_index[0].reshape(e // EW, EW)
    dst2 = edge_index[1].reshape(e // EW, EW)

    degS = _sc_edge(jnp.ones((1, n_pad, 128), jnp.float32), src2, dst2)
    dinv16 = _dinv16(degS, n)

    h = x
    for W, b in ((W1, b1), (W2, b2), (W3, b3)):
        g = _matmul_g(h, W, dinv16, n_pad)
        S = _sc_edge(g, src2, dst2)
        h = _activate(S, dinv16, b.reshape(-1, 1, 128), n)
    return h
